# Initial kernel scaffold; baseline (speedup 1.0000x reference)
import jax
import jax.numpy as jnp
from jax.experimental import pallas as pl


def kernel(predicted_target, target):
    raise NotImplementedError("write your pallas kernel here")


# speedup vs baseline: 1.8133x; 1.8133x over previous
import functools
import math

import jax
import jax.numpy as jnp
from jax.experimental import pallas as pl
from jax.experimental.pallas import tpu as pltpu

_SMOOTH = 0.1
_PAD = 0
_BV = 4096
_LW = 512


def _loss_body(p_ref, t_ref, out_ref, acc_ref, pt_ref, p0_ref, *, n_rows, vocab):
    i = pl.program_id(0)
    nb = pl.num_programs(0)

    @pl.when(i == 0)
    def _init():
        acc_ref[...] = jnp.zeros_like(acc_ref)
        pt_ref[...] = jnp.zeros_like(pt_ref)
        p0_ref[...] = p_ref[:, 0:1]

    t = t_ref[...]
    base = i * _BV

    def _accumulate(masked):
        for k in range(_BV // _LW):
            x = p_ref[:, k * _LW:(k + 1) * _LW]
            col = (base + k * _LW) + jax.lax.broadcasted_iota(
                jnp.int32, (n_rows, _LW), 1)
            if masked:
                x = jnp.where(col < vocab, x, 0.0)
            acc_ref[...] += x
            pt_ref[...] += jnp.where(col == t, x, 0.0)

    @pl.when(i < nb - 1)
    def _main():
        _accumulate(masked=False)

    @pl.when(i == nb - 1)
    def _last():
        _accumulate(masked=True)
        eps = _SMOOTH / (vocab - 2)
        conf = 1.0 - _SMOOTH
        kconst = conf * math.log(conf) + (vocab - 2) * eps * math.log(eps)
        s = jnp.sum(acc_ref[...], axis=1, keepdims=True)
        pt = jnp.sum(pt_ref[...], axis=1, keepdims=True)
        p0 = p0_ref[...]
        row = jnp.float32(kconst) - jnp.float32(eps) * (s - p0 - pt) \
            - jnp.float32(conf) * pt
        masked = jnp.where(t != _PAD, row, 0.0)
        out_ref[...] = jnp.sum(masked, axis=0, keepdims=True)


def kernel(predicted_target, target):
    n, v = predicted_target.shape
    nb = (v + _BV - 1) // _BV
    t2 = target.reshape(n, 1)
    out = pl.pallas_call(
        functools.partial(_loss_body, n_rows=n, vocab=v),
        grid=(nb,),
        in_specs=[
            pl.BlockSpec((n, _BV), lambda i: (0, i)),
            pl.BlockSpec((n, 1), lambda i: (0, 0)),
        ],
        out_specs=pl.BlockSpec((1, 1), lambda i: (0, 0)),
        out_shape=jax.ShapeDtypeStruct((1, 1), jnp.float32),
        scratch_shapes=[
            pltpu.VMEM((n, _LW), jnp.float32),
            pltpu.VMEM((n, _LW), jnp.float32),
            pltpu.VMEM((n, 1), jnp.float32),
        ],
        compiler_params=pltpu.CompilerParams(
            dimension_semantics=("arbitrary",)),
    )(predicted_target, t2)
    return out[0, 0]
